# baseline (speedup 1.0000x reference)
import functools

import jax
import jax.numpy as jnp
from jax import lax
from jax.experimental import pallas as pl
from jax.experimental.pallas import tpu as pltpu
from jax.experimental.pallas import tpu_sc as plsc

_NW = 32
_CELLS = 64 * 10647
_CPW = _CELLS // _NW
_CT = 1183
_NT = _CPW // _CT
_L = 16
_NG = (_CT + _L - 1) // _L


def _sc_body(x_hbm, y_hbm, o_hbm, xa, ya, xb, yb, oacc, sema, semb):
    wid = lax.axis_index("s") * 2 + lax.axis_index("c")
    lanes = lax.broadcasted_iota(jnp.int32, (_L,), 0)

    row0 = wid * _NT

    def start(t, xbuf, ybuf, sem):
        pltpu.async_copy(x_hbm.at[row0 + t], xbuf, sem)
        pltpu.async_copy(y_hbm.at[row0 + t], ybuf, sem)

    def wait(xbuf, ybuf, sem):
        pltpu.make_async_copy(x_hbm.at[row0], xbuf, sem).wait()
        pltpu.make_async_copy(y_hbm.at[row0], ybuf, sem).wait()

    def compute(xbuf, ybuf, acc):
        def group(g, acc):
            cells_raw = g * _L + lanes
            valid = cells_raw < _CT
            cells = jnp.where(valid, cells_raw, 0)
            base = cells * 25
            yv0 = plsc.load_gather(ybuf, [base])
            xv0 = plsc.load_gather(xbuf, [base])
            m = yv0 > 0.5
            d0 = yv0 - xv0
            s = d0 * d0
            for c in range(1, 25):
                idx = base + c
                xv = plsc.load_gather(xbuf, [idx])
                yv = plsc.load_gather(ybuf, [idx])
                d = yv - xv
                s = s + d * d
            contrib = jnp.where(m, s, 0.1 * (xv0 * xv0))
            contrib = jnp.where(valid, contrib, 0.0)
            return acc + contrib

        return lax.fori_loop(0, _NG, group, acc)

    acc = jnp.zeros((_L,), jnp.float32)
    start(0, xa, ya, sema)

    def pair(k, acc):
        t0 = 2 * k
        start(t0 + 1, xb, yb, semb)
        wait(xa, ya, sema)
        acc = compute(xa, ya, acc)

        @pl.when(k < _NT // 2 - 1)
        def _():
            start(t0 + 2, xa, ya, sema)

        wait(xb, yb, semb)
        acc = compute(xb, yb, acc)
        return acc

    acc = lax.fori_loop(0, _NT // 2, pair, acc)
    oacc[...] = acc
    pltpu.sync_copy(oacc, o_hbm.at[wid])


_sc_call = pl.kernel(
    _sc_body,
    out_type=jax.ShapeDtypeStruct((_NW, _L), jnp.float32),
    mesh=plsc.VectorSubcoreMesh(core_axis_name="c", subcore_axis_name="s"),
    scratch_types=[
        pltpu.VMEM((_CT * 25,), jnp.float32),
        pltpu.VMEM((_CT * 25,), jnp.float32),
        pltpu.VMEM((_CT * 25,), jnp.float32),
        pltpu.VMEM((_CT * 25,), jnp.float32),
        pltpu.VMEM((_L,), jnp.float32),
        pltpu.SemaphoreType.DMA,
        pltpu.SemaphoreType.DMA,
    ],
    compiler_params=pltpu.CompilerParams(
        use_tc_tiling_on_sc=False, needs_layout_passes=False),
)


def kernel(x, y):
    xr = x.reshape(_NW * _NT, _CT * 25)
    yr = y.reshape(_NW * _NT, _CT * 25)
    partials = _sc_call(xr, yr)
    return jnp.sum(partials)
